# initial kernel scaffold (unmeasured)
import jax
import jax.numpy as jnp
from jax import lax
from jax.experimental import pallas as pl
from jax.experimental.pallas import tpu as pltpu

N_DEV = 4
TW = 2048


def _any_space():
    if hasattr(pltpu, "ANY"):
        return pltpu.ANY
    return pltpu.MemorySpace.ANY


def _rs_silu(partial, m_per):
    m, n = partial.shape
    nt = n // TW
    ANY = _any_space()

    def body(p_ref, out_ref, send_hbm, recv_hbm, va, vb, send_sem, recv_sems, dma_sem):
        my = lax.axis_index("i")
        left = lax.rem(my + N_DEV - 1, N_DEV)
        right = lax.rem(my + 1, N_DEV)

        barrier_sem = pltpu.get_barrier_semaphore()
        for nbr in (left, right):
            pl.semaphore_signal(
                barrier_sem, inc=1,
                device_id=(nbr,), device_id_type=pl.DeviceIdType.MESH,
            )
        pl.semaphore_wait(barrier_sem, 2)

        def copy(src, dst):
            cp = pltpu.make_async_copy(src, dst, dma_sem)
            cp.start()
            cp.wait()

        for h in range(N_DEV - 1):
            c = lax.rem(my + 2 * N_DEV - 1 - h, N_DEV)
            recv_slot = h % 2
            prev_slot = (h - 1) % 2
            for t in range(nt):
                cols = pl.ds(t * TW, TW)
                copy(p_ref.at[pl.ds(c * m_per, m_per), cols], va)
                if h > 0:
                    copy(recv_hbm.at[prev_slot, :, cols], vb)
                    va[...] = va[...] + vb[...]
                copy(va, send_hbm.at[:, cols])
            rdma = pltpu.make_async_remote_copy(
                src_ref=send_hbm,
                dst_ref=recv_hbm.at[recv_slot],
                send_sem=send_sem,
                recv_sem=recv_sems.at[recv_slot],
                device_id=(right,),
                device_id_type=pl.DeviceIdType.MESH,
            )
            rdma.start()
            rdma.wait()

        last_slot = (N_DEV - 2) % 2
        for t in range(nt):
            cols = pl.ds(t * TW, TW)
            copy(p_ref.at[pl.ds(my * m_per, m_per), cols], va)
            copy(recv_hbm.at[last_slot, :, cols], vb)
            y = va[...] + vb[...]
            out_ref[:, cols] = y * jax.nn.sigmoid(y)

    return pl.pallas_call(
        body,
        out_shape=jax.ShapeDtypeStruct((m_per, n), jnp.float32),
        in_specs=[pl.BlockSpec(memory_space=ANY)],
        out_specs=pl.BlockSpec(memory_space=pltpu.VMEM),
        scratch_shapes=[
            ANY((m_per, n), jnp.float32),
            ANY((2, m_per, n), jnp.float32),
            pltpu.VMEM((m_per, TW), jnp.float32),
            pltpu.VMEM((m_per, TW), jnp.float32),
            pltpu.SemaphoreType.DMA,
            pltpu.SemaphoreType.DMA((2,)),
            pltpu.SemaphoreType.DMA,
        ],
        compiler_params=pltpu.CompilerParams(collective_id=0),
    )(partial)


def kernel(x, w_mat):
    m = x.shape[0]
    m_per = m // N_DEV
    partial = jnp.dot(x, w_mat, preferred_element_type=jnp.float32)
    return _rs_silu(partial, m_per)


# baseline (device time: 1356233 ns/iter reference)
import jax
import jax.numpy as jnp
from jax import lax
from jax.experimental import pallas as pl
from jax.experimental.pallas import tpu as pltpu

N_DEV = 4
TW = 2048


def _rs_silu(partial, m_per):
    m, n = partial.shape
    nt = n // TW

    def body(p_ref, out_ref, send_hbm, recv_hbm, va, vb, send_sem, recv_sems, dma_sem):
        my = lax.axis_index("i")
        left = lax.rem(my + N_DEV - 1, N_DEV)
        right = lax.rem(my + 1, N_DEV)

        barrier_sem = pltpu.get_barrier_semaphore()
        for nbr in (left, right):
            pl.semaphore_signal(
                barrier_sem, inc=1,
                device_id=(nbr,), device_id_type=pl.DeviceIdType.MESH,
            )
        pl.semaphore_wait(barrier_sem, 2)

        def copy(src, dst):
            cp = pltpu.make_async_copy(src, dst, dma_sem)
            cp.start()
            cp.wait()

        for h in range(N_DEV - 1):
            c = lax.rem(my + 2 * N_DEV - 1 - h, N_DEV)
            recv_slot = h % 2
            prev_slot = (h - 1) % 2
            for t in range(nt):
                cols = pl.ds(t * TW, TW)
                copy(p_ref.at[pl.ds(c * m_per, m_per), cols], va)
                if h > 0:
                    copy(recv_hbm.at[prev_slot, :, cols], vb)
                    va[...] = va[...] + vb[...]
                copy(va, send_hbm.at[:, cols])
            rdma = pltpu.make_async_remote_copy(
                src_ref=send_hbm,
                dst_ref=recv_hbm.at[recv_slot],
                send_sem=send_sem,
                recv_sem=recv_sems.at[recv_slot],
                device_id=(right,),
                device_id_type=pl.DeviceIdType.MESH,
            )
            rdma.start()
            rdma.wait()

        last_slot = (N_DEV - 2) % 2
        for t in range(nt):
            cols = pl.ds(t * TW, TW)
            copy(p_ref.at[pl.ds(my * m_per, m_per), cols], va)
            copy(recv_hbm.at[last_slot, :, cols], vb)
            y = va[...] + vb[...]
            va[...] = y * jax.nn.sigmoid(y)
            copy(va, out_ref.at[:, cols])

    out, _, _ = pl.pallas_call(
        body,
        out_shape=[
            jax.ShapeDtypeStruct((m_per, n), jnp.float32),
            jax.ShapeDtypeStruct((m_per, n), jnp.float32),
            jax.ShapeDtypeStruct((2, m_per, n), jnp.float32),
        ],
        in_specs=[pl.BlockSpec(memory_space=pl.ANY)],
        out_specs=[
            pl.BlockSpec(memory_space=pl.ANY),
            pl.BlockSpec(memory_space=pl.ANY),
            pl.BlockSpec(memory_space=pl.ANY),
        ],
        scratch_shapes=[
            pltpu.VMEM((m_per, TW), jnp.float32),
            pltpu.VMEM((m_per, TW), jnp.float32),
            pltpu.SemaphoreType.DMA,
            pltpu.SemaphoreType.DMA((2,)),
            pltpu.SemaphoreType.DMA,
        ],
        compiler_params=pltpu.CompilerParams(collective_id=0),
    )(partial)
    return out


def kernel(x, w_mat):
    m = x.shape[0]
    m_per = m // N_DEV
    partial = jnp.dot(x, w_mat, preferred_element_type=jnp.float32)
    return _rs_silu(partial, m_per)


# device time: 745700 ns/iter; 1.8187x vs baseline; 1.8187x over previous
import jax
import jax.numpy as jnp
from jax import lax
from jax.experimental import pallas as pl
from jax.experimental.pallas import tpu as pltpu

N_DEV = 4
TS = 1024
NT = 4


def _rs_silu(partial, m_per):
    m, n = partial.shape
    half = n // 2
    assert half == NT * TS

    def body(p_ref, out_ref, recv_cw, recv_ccw, s_cw, s_ccw, vb,
             send_sems_cw, send_sems_ccw, recv_sems_cw, recv_sems_ccw,
             dma_a):
        my = lax.axis_index("i")
        left = lax.rem(my + N_DEV - 1, N_DEV)
        right = lax.rem(my + 1, N_DEV)

        barrier_sem = pltpu.get_barrier_semaphore()
        for nbr in (left, right):
            pl.semaphore_signal(
                barrier_sem, inc=1,
                device_id=(nbr,), device_id_type=pl.DeviceIdType.MESH,
            )
        pl.semaphore_wait(barrier_sem, 2)

        def copy(src, dst, sem):
            cp = pltpu.make_async_copy(src, dst, sem)
            cp.start()
            return cp

        dirs = (
            (s_cw, recv_cw, send_sems_cw, recv_sems_cw, 0, right),
            (s_ccw, recv_ccw, send_sems_ccw, recv_sems_ccw, half, left),
        )

        for h in range(N_DEV - 1):
            c_cw = lax.rem(my + 2 * N_DEV - 1 - h, N_DEV)
            c_ccw = lax.rem(my + 1 + h, N_DEV)
            slot = h % 2
            prev = (h - 1) % 2
            rdmas = []
            for t in range(NT):
                for (s_buf, r_buf, s_sems, r_sems, col0, peer), c in zip(
                        dirs, (c_cw, c_ccw)):
                    cols = pl.ds(col0 + t * TS, TS)
                    copy(p_ref.at[pl.ds(c * m_per, m_per), cols],
                         s_buf.at[t], dma_a).wait()
                    if h > 0:
                        copy(r_buf.at[prev, :, pl.ds(t * TS, TS)],
                             vb, dma_a).wait()
                        s_buf[t] = s_buf[t] + vb[...]
                    rdma = pltpu.make_async_remote_copy(
                        src_ref=s_buf.at[t],
                        dst_ref=r_buf.at[slot, :, pl.ds(t * TS, TS)],
                        send_sem=s_sems.at[t],
                        recv_sem=r_sems.at[slot, t],
                        device_id=(peer,),
                        device_id_type=pl.DeviceIdType.MESH,
                    )
                    rdma.start()
                    rdmas.append(rdma)
            for rdma in rdmas:
                rdma.wait()

        last = (N_DEV - 2) % 2
        for t in range(NT):
            for s_buf, r_buf, _, _, col0, _ in dirs:
                cols = pl.ds(col0 + t * TS, TS)
                copy(p_ref.at[pl.ds(my * m_per, m_per), cols],
                     s_buf.at[t], dma_a).wait()
                copy(r_buf.at[last, :, pl.ds(t * TS, TS)], vb, dma_a).wait()
                y = s_buf[t] + vb[...]
                s_buf[t] = y * jax.nn.sigmoid(y)
                copy(s_buf.at[t], out_ref.at[:, cols], dma_a).wait()

    out, _, _ = pl.pallas_call(
        body,
        out_shape=[
            jax.ShapeDtypeStruct((m_per, n), jnp.float32),
            jax.ShapeDtypeStruct((2, m_per, half), jnp.float32),
            jax.ShapeDtypeStruct((2, m_per, half), jnp.float32),
        ],
        in_specs=[pl.BlockSpec(memory_space=pl.ANY)],
        out_specs=[
            pl.BlockSpec(memory_space=pl.ANY),
            pl.BlockSpec(memory_space=pl.ANY),
            pl.BlockSpec(memory_space=pl.ANY),
        ],
        scratch_shapes=[
            pltpu.VMEM((NT, m_per, TS), jnp.float32),
            pltpu.VMEM((NT, m_per, TS), jnp.float32),
            pltpu.VMEM((m_per, TS), jnp.float32),
            pltpu.SemaphoreType.DMA((NT,)),
            pltpu.SemaphoreType.DMA((NT,)),
            pltpu.SemaphoreType.DMA((2, NT)),
            pltpu.SemaphoreType.DMA((2, NT)),
            pltpu.SemaphoreType.DMA,
        ],
        compiler_params=pltpu.CompilerParams(
            collective_id=0, vmem_limit_bytes=63 * 1024 * 1024,
        ),
    )(partial)
    return out


def kernel(x, w_mat):
    m = x.shape[0]
    m_per = m // N_DEV
    partial = jnp.dot(x, w_mat, preferred_element_type=jnp.float32)
    return _rs_silu(partial, m_per)


# device time: 674223 ns/iter; 2.0115x vs baseline; 1.1060x over previous
import jax
import jax.numpy as jnp
from jax import lax
from jax.experimental import pallas as pl
from jax.experimental.pallas import tpu as pltpu

N_DEV = 4
TS = 1024
NT = 4


def _rs_silu(partial, m_per):
    m, n = partial.shape
    half = n // 2
    assert half == NT * TS

    def body(p_ref, out_ref, recv_cw, recv_ccw, s_cw, s_ccw, vb,
             send_sems_cw, send_sems_ccw, recv_sems_cw, recv_sems_ccw,
             dma_a, dma_b):
        my = lax.axis_index("i")
        left = lax.rem(my + N_DEV - 1, N_DEV)
        right = lax.rem(my + 1, N_DEV)

        barrier_sem = pltpu.get_barrier_semaphore()
        for nbr in (left, right):
            pl.semaphore_signal(
                barrier_sem, inc=1,
                device_id=(nbr,), device_id_type=pl.DeviceIdType.MESH,
            )
        pl.semaphore_wait(barrier_sem, 2)

        def copy(src, dst, sem):
            cp = pltpu.make_async_copy(src, dst, sem)
            cp.start()
            return cp

        dirs = (
            (s_cw, recv_cw, send_sems_cw, recv_sems_cw, 0, right),
            (s_ccw, recv_ccw, send_sems_ccw, recv_sems_ccw, half, left),
        )

        inflight = [[None] * NT for _ in dirs]
        for h in range(N_DEV - 1):
            c_cw = lax.rem(my + 2 * N_DEV - 1 - h, N_DEV)
            c_ccw = lax.rem(my + 1 + h, N_DEV)
            slot = h % 2
            prev = (h - 1) % 2
            for t in range(NT):
                for d, ((s_buf, r_buf, s_sems, r_sems, col0, peer), c) in (
                        enumerate(zip(dirs, (c_cw, c_ccw)))):
                    if h > 0:
                        inflight[d][t].wait()
                    cols = pl.ds(col0 + t * TS, TS)
                    cp_p = copy(p_ref.at[pl.ds(c * m_per, m_per), cols],
                                s_buf.at[t], dma_a)
                    if h > 0:
                        copy(r_buf.at[prev, :, pl.ds(t * TS, TS)],
                             vb, dma_b).wait()
                    cp_p.wait()
                    if h > 0:
                        s_buf[t] = s_buf[t] + vb[...]
                    rdma = pltpu.make_async_remote_copy(
                        src_ref=s_buf.at[t],
                        dst_ref=r_buf.at[slot, :, pl.ds(t * TS, TS)],
                        send_sem=s_sems.at[t],
                        recv_sem=r_sems.at[slot, t],
                        device_id=(peer,),
                        device_id_type=pl.DeviceIdType.MESH,
                    )
                    rdma.start()
                    inflight[d][t] = rdma

        last = (N_DEV - 2) % 2
        for t in range(NT):
            for d, (s_buf, r_buf, _, _, col0, _) in enumerate(dirs):
                inflight[d][t].wait()
                cols = pl.ds(col0 + t * TS, TS)
                cp_p = copy(p_ref.at[pl.ds(my * m_per, m_per), cols],
                            s_buf.at[t], dma_a)
                copy(r_buf.at[last, :, pl.ds(t * TS, TS)], vb, dma_b).wait()
                cp_p.wait()
                y = s_buf[t] + vb[...]
                s_buf[t] = y * jax.nn.sigmoid(y)
                copy(s_buf.at[t], out_ref.at[:, cols], dma_a).wait()

    out, _, _ = pl.pallas_call(
        body,
        out_shape=[
            jax.ShapeDtypeStruct((m_per, n), jnp.float32),
            jax.ShapeDtypeStruct((2, m_per, half), jnp.float32),
            jax.ShapeDtypeStruct((2, m_per, half), jnp.float32),
        ],
        in_specs=[pl.BlockSpec(memory_space=pl.ANY)],
        out_specs=[
            pl.BlockSpec(memory_space=pl.ANY),
            pl.BlockSpec(memory_space=pl.ANY),
            pl.BlockSpec(memory_space=pl.ANY),
        ],
        scratch_shapes=[
            pltpu.VMEM((NT, m_per, TS), jnp.float32),
            pltpu.VMEM((NT, m_per, TS), jnp.float32),
            pltpu.VMEM((m_per, TS), jnp.float32),
            pltpu.SemaphoreType.DMA((NT,)),
            pltpu.SemaphoreType.DMA((NT,)),
            pltpu.SemaphoreType.DMA((2, NT)),
            pltpu.SemaphoreType.DMA((2, NT)),
            pltpu.SemaphoreType.DMA,
            pltpu.SemaphoreType.DMA,
        ],
        compiler_params=pltpu.CompilerParams(
            collective_id=0, vmem_limit_bytes=63 * 1024 * 1024,
        ),
    )(partial)
    return out


def kernel(x, w_mat):
    m = x.shape[0]
    m_per = m // N_DEV
    partial = jnp.dot(x, w_mat, preferred_element_type=jnp.float32)
    return _rs_silu(partial, m_per)


# device time: 623504 ns/iter; 2.1752x vs baseline; 1.0813x over previous
import jax
import jax.numpy as jnp
from jax import lax
from jax.experimental import pallas as pl
from jax.experimental.pallas import tpu as pltpu

N_DEV = 4
TS = 1024
NT = 4


def _fused(x, w_mat, m_per):
    m, k_per = x.shape
    n = w_mat.shape[1]
    half = n // 2
    assert half == NT * TS
    n_hops = N_DEV - 1

    def body(x_ref, w_ref, out_ref, recv_cw, recv_ccw, xv, wt, s_cw, s_ccw,
             vb, send_sems_cw, send_sems_ccw, recv_sems_cw, recv_sems_ccw,
             dma_a, dma_b):
        my = lax.axis_index("i")
        left = lax.rem(my + N_DEV - 1, N_DEV)
        right = lax.rem(my + 1, N_DEV)

        cpx = pltpu.make_async_copy(x_ref, xv, dma_a)
        cpx.start()

        barrier_sem = pltpu.get_barrier_semaphore()
        for nbr in (left, right):
            pl.semaphore_signal(
                barrier_sem, inc=1,
                device_id=(nbr,), device_id_type=pl.DeviceIdType.MESH,
            )
        pl.semaphore_wait(barrier_sem, 2)
        cpx.wait()

        def copy(src, dst, sem):
            cp = pltpu.make_async_copy(src, dst, sem)
            cp.start()
            return cp

        dirs = (
            (s_cw, recv_cw, send_sems_cw, recv_sems_cw, 0, right),
            (s_ccw, recv_ccw, send_sems_ccw, recv_sems_ccw, half, left),
        )
        descs = [[], []]

        for h in range(n_hops):
            c_cw = lax.rem(my + 2 * N_DEV - 1 - h, N_DEV)
            c_ccw = lax.rem(my + 1 + h, N_DEV)
            slot = h % 2
            prev = (h - 1) % 2
            for t in range(NT):
                for d, ((s_buf, r_buf, s_sems, r_sems, col0, peer), c) in (
                        enumerate(zip(dirs, (c_cw, c_ccw)))):
                    kk = h * NT + t
                    sslot = kk % 2
                    if kk >= 2:
                        descs[d][kk - 2].wait_send()
                    cpw = copy(w_ref.at[:, pl.ds(col0 + t * TS, TS)],
                               wt, dma_a)
                    if h > 0:
                        descs[d][kk - NT].wait_recv()
                        cpr = copy(r_buf.at[prev, :, pl.ds(t * TS, TS)],
                                   vb, dma_b)
                    cpw.wait()
                    xa = xv[pl.ds(c * m_per, m_per), :]
                    if h > 0:
                        cpr.wait()
                        s_buf[sslot] = jnp.dot(
                            xa, wt[...], preferred_element_type=jnp.float32
                        ) + vb[...]
                    else:
                        s_buf[sslot] = jnp.dot(
                            xa, wt[...], preferred_element_type=jnp.float32
                        )
                    rdma = pltpu.make_async_remote_copy(
                        src_ref=s_buf.at[sslot],
                        dst_ref=r_buf.at[slot, :, pl.ds(t * TS, TS)],
                        send_sem=s_sems.at[sslot],
                        recv_sem=r_sems.at[slot, t],
                        device_id=(peer,),
                        device_id_type=pl.DeviceIdType.MESH,
                    )
                    rdma.start()
                    descs[d].append(rdma)

        last = (n_hops - 1) % 2
        for t in range(NT):
            for d, (s_buf, r_buf, s_sems, r_sems, col0, peer) in (
                    enumerate(dirs)):
                kk = n_hops * NT + t
                if kk - 2 < len(descs[d]):
                    descs[d][kk - 2].wait_send()
                descs[d][(n_hops - 1) * NT + t].wait_recv()
                cols = pl.ds(col0 + t * TS, TS)
                cpw = copy(w_ref.at[:, cols], wt, dma_a)
                cpr = copy(r_buf.at[last, :, pl.ds(t * TS, TS)], vb, dma_b)
                cpw.wait()
                cpr.wait()
                xa = xv[pl.ds(my * m_per, m_per), :]
                y = jnp.dot(
                    xa, wt[...], preferred_element_type=jnp.float32
                ) + vb[...]
                vb[...] = y * jax.nn.sigmoid(y)
                copy(vb, out_ref.at[:, cols], dma_b).wait()

    out, _, _ = pl.pallas_call(
        body,
        out_shape=[
            jax.ShapeDtypeStruct((m_per, n), jnp.float32),
            jax.ShapeDtypeStruct((2, m_per, half), jnp.float32),
            jax.ShapeDtypeStruct((2, m_per, half), jnp.float32),
        ],
        in_specs=[
            pl.BlockSpec(memory_space=pl.ANY),
            pl.BlockSpec(memory_space=pl.ANY),
        ],
        out_specs=[
            pl.BlockSpec(memory_space=pl.ANY),
            pl.BlockSpec(memory_space=pl.ANY),
            pl.BlockSpec(memory_space=pl.ANY),
        ],
        scratch_shapes=[
            pltpu.VMEM((m, k_per), jnp.float32),
            pltpu.VMEM((k_per, TS), jnp.float32),
            pltpu.VMEM((2, m_per, TS), jnp.float32),
            pltpu.VMEM((2, m_per, TS), jnp.float32),
            pltpu.VMEM((m_per, TS), jnp.float32),
            pltpu.SemaphoreType.DMA((2,)),
            pltpu.SemaphoreType.DMA((2,)),
            pltpu.SemaphoreType.DMA((2, NT)),
            pltpu.SemaphoreType.DMA((2, NT)),
            pltpu.SemaphoreType.DMA,
            pltpu.SemaphoreType.DMA,
        ],
        compiler_params=pltpu.CompilerParams(
            collective_id=0, vmem_limit_bytes=63 * 1024 * 1024,
        ),
    )(x, w_mat)
    return out


def kernel(x, w_mat):
    m = x.shape[0]
    m_per = m // N_DEV
    return _fused(x, w_mat, m_per)


# device time: 623054 ns/iter; 2.1768x vs baseline; 1.0007x over previous
import jax
import jax.numpy as jnp
from jax import lax
from jax.experimental import pallas as pl
from jax.experimental.pallas import tpu as pltpu

N_DEV = 4
TS = 1024
NT = 4


def _fused(x, w_mat, m_per):
    m, k_per = x.shape
    n = w_mat.shape[1]
    half = n // 2
    assert half == NT * TS
    n_hops = N_DEV - 1

    def body(x_ref, w_ref, out_ref, recv_cw, recv_ccw, xv, wt, s_cw, s_ccw,
             vb, send_sems_cw, send_sems_ccw, recv_sems_cw, recv_sems_ccw,
             dma_a, dma_b):
        my = lax.axis_index("i")
        left = lax.rem(my + N_DEV - 1, N_DEV)
        right = lax.rem(my + 1, N_DEV)

        cpx = pltpu.make_async_copy(x_ref, xv, dma_a)
        cpx.start()

        barrier_sem = pltpu.get_barrier_semaphore()
        for nbr in (left, right):
            pl.semaphore_signal(
                barrier_sem, inc=1,
                device_id=(nbr,), device_id_type=pl.DeviceIdType.MESH,
            )
        pl.semaphore_wait(barrier_sem, 2)
        cpx.wait()

        def copy(src, dst, sem):
            cp = pltpu.make_async_copy(src, dst, sem)
            cp.start()
            return cp

        dirs = (
            (s_cw, recv_cw, send_sems_cw, recv_sems_cw, 0, right),
            (s_ccw, recv_ccw, send_sems_ccw, recv_sems_ccw, half, left),
        )
        descs = [[], []]

        for h in range(n_hops):
            c_cw = lax.rem(my + 2 * N_DEV - 1 - h, N_DEV)
            c_ccw = lax.rem(my + 1 + h, N_DEV)
            slot = h % 2
            prev = (h - 1) % 2
            for t in range(NT):
                for d, ((s_buf, r_buf, s_sems, r_sems, col0, peer), c) in (
                        enumerate(zip(dirs, (c_cw, c_ccw)))):
                    kk = h * NT + t
                    sslot = kk % 2
                    if kk >= 2:
                        descs[d][kk - 2].wait_send()
                    cpw = copy(w_ref.at[:, pl.ds(col0 + t * TS, TS)],
                               wt, dma_a)
                    if h > 0:
                        descs[d][kk - NT].wait_recv()
                        cpr = copy(r_buf.at[prev, :, pl.ds(t * TS, TS)],
                                   vb, dma_b)
                    cpw.wait()
                    xa = xv[pl.ds(c * m_per, m_per), :].astype(jnp.bfloat16)
                    wb = wt[...].astype(jnp.bfloat16)
                    if h > 0:
                        cpr.wait()
                        s_buf[sslot] = jnp.dot(
                            xa, wb, preferred_element_type=jnp.float32
                        ) + vb[...]
                    else:
                        s_buf[sslot] = jnp.dot(
                            xa, wb, preferred_element_type=jnp.float32
                        )
                    rdma = pltpu.make_async_remote_copy(
                        src_ref=s_buf.at[sslot],
                        dst_ref=r_buf.at[slot, :, pl.ds(t * TS, TS)],
                        send_sem=s_sems.at[sslot],
                        recv_sem=r_sems.at[slot, t],
                        device_id=(peer,),
                        device_id_type=pl.DeviceIdType.MESH,
                    )
                    rdma.start()
                    descs[d].append(rdma)

        last = (n_hops - 1) % 2
        for t in range(NT):
            for d, (s_buf, r_buf, s_sems, r_sems, col0, peer) in (
                    enumerate(dirs)):
                kk = n_hops * NT + t
                if kk - 2 < len(descs[d]):
                    descs[d][kk - 2].wait_send()
                descs[d][(n_hops - 1) * NT + t].wait_recv()
                cols = pl.ds(col0 + t * TS, TS)
                cpw = copy(w_ref.at[:, cols], wt, dma_a)
                cpr = copy(r_buf.at[last, :, pl.ds(t * TS, TS)], vb, dma_b)
                cpw.wait()
                cpr.wait()
                xa = xv[pl.ds(my * m_per, m_per), :].astype(jnp.bfloat16)
                y = jnp.dot(
                    xa, wt[...].astype(jnp.bfloat16),
                    preferred_element_type=jnp.float32,
                ) + vb[...]
                vb[...] = y * jax.nn.sigmoid(y)
                copy(vb, out_ref.at[:, cols], dma_b).wait()

    out, _, _ = pl.pallas_call(
        body,
        out_shape=[
            jax.ShapeDtypeStruct((m_per, n), jnp.float32),
            jax.ShapeDtypeStruct((2, m_per, half), jnp.float32),
            jax.ShapeDtypeStruct((2, m_per, half), jnp.float32),
        ],
        in_specs=[
            pl.BlockSpec(memory_space=pl.ANY),
            pl.BlockSpec(memory_space=pl.ANY),
        ],
        out_specs=[
            pl.BlockSpec(memory_space=pl.ANY),
            pl.BlockSpec(memory_space=pl.ANY),
            pl.BlockSpec(memory_space=pl.ANY),
        ],
        scratch_shapes=[
            pltpu.VMEM((m, k_per), jnp.float32),
            pltpu.VMEM((k_per, TS), jnp.float32),
            pltpu.VMEM((2, m_per, TS), jnp.float32),
            pltpu.VMEM((2, m_per, TS), jnp.float32),
            pltpu.VMEM((m_per, TS), jnp.float32),
            pltpu.SemaphoreType.DMA((2,)),
            pltpu.SemaphoreType.DMA((2,)),
            pltpu.SemaphoreType.DMA((2, NT)),
            pltpu.SemaphoreType.DMA((2, NT)),
            pltpu.SemaphoreType.DMA,
            pltpu.SemaphoreType.DMA,
        ],
        compiler_params=pltpu.CompilerParams(
            collective_id=0, vmem_limit_bytes=63 * 1024 * 1024,
        ),
    )(x, w_mat)
    return out


def kernel(x, w_mat):
    m = x.shape[0]
    m_per = m // N_DEV
    return _fused(x, w_mat, m_per)
